# Initial kernel scaffold; baseline (speedup 1.0000x reference)
import jax
import jax.numpy as jnp
from jax.experimental import pallas as pl


def kernel(tokenid, table):
    raise NotImplementedError("write your pallas kernel here")


# speedup vs baseline: 5.7118x; 5.7118x over previous
import jax
import jax.numpy as jnp
from jax import lax
from jax.experimental import pallas as pl
from jax.experimental.pallas import tpu as pltpu
from jax.experimental.pallas import tpu_sc as plsc

BATCH = 1024
SEQLEN = 200
D = 128
B = BATCH * SEQLEN
NC = 2
NS = 16
NW = NC * NS
CHUNK = 128
NCHUNK = B // (NW * CHUNK)


def _emb_body(idx_hbm, table_hbm, out_hbm, idx_v, rows_v, sem):
    wid = lax.axis_index("s") * NC + lax.axis_index("c")
    pltpu.sync_copy(idx_hbm.at[wid], idx_v)

    def chunk_body(c, carry):
        pltpu.async_copy(table_hbm.at[idx_v.at[c]], rows_v, sem).wait()
        pltpu.sync_copy(rows_v, out_hbm.at[wid, c])
        return carry

    lax.fori_loop(0, NCHUNK, chunk_body, 0)


_emb_call = pl.kernel(
    _emb_body,
    out_type=jax.ShapeDtypeStruct((NW, NCHUNK, CHUNK, D), jnp.float32),
    mesh=plsc.VectorSubcoreMesh(core_axis_name="c", subcore_axis_name="s"),
    scratch_types=[
        pltpu.VMEM((NCHUNK, CHUNK), jnp.int32),
        pltpu.VMEM((CHUNK, D), jnp.float32),
        pltpu.SemaphoreType.DMA,
    ],
)


def _mask_body(tok_ref, m_ref):
    m_ref[...] = (tok_ref[...] > 0).astype(jnp.int8)


_mask_call = pl.pallas_call(
    _mask_body,
    out_shape=jax.ShapeDtypeStruct((BATCH, SEQLEN), jnp.int8),
)


def kernel(tokenid, table):
    idx3 = tokenid.reshape(NW, NCHUNK, CHUNK)
    emb = _emb_call(idx3, table)
    input_emb = emb.reshape(BATCH, SEQLEN, D)
    mask = _mask_call(tokenid).astype(jnp.bool_)
    return (input_emb, mask)


# speedup vs baseline: 7.8249x; 1.3699x over previous
import jax
import jax.numpy as jnp
from jax import lax
from jax.experimental import pallas as pl
from jax.experimental.pallas import tpu as pltpu
from jax.experimental.pallas import tpu_sc as plsc

BATCH = 1024
SEQLEN = 200
D = 128
B = BATCH * SEQLEN
NC = 2
NS = 16
NW = NC * NS
CHUNK = 128
NCHUNK = B // (NW * CHUNK)


NBUF = 2


def _emb_body(idx_hbm, table_hbm, out_hbm, idx_v, rows_v, gsem, ssem):
    wid = lax.axis_index("s") * NC + lax.axis_index("c")
    pltpu.sync_copy(idx_hbm.at[wid], idx_v)

    def start_gather(g, b):
        pltpu.make_async_copy(
            table_hbm.at[idx_v.at[g]], rows_v.at[b], gsem.at[b]).start()

    def wait_gather(b):
        pltpu.make_async_copy(
            table_hbm.at[idx_v.at[0]], rows_v.at[b], gsem.at[b]).wait()

    def start_scatter(g, b):
        pltpu.make_async_copy(
            rows_v.at[b], out_hbm.at[wid, g], ssem.at[b]).start()

    def wait_scatter(b):
        pltpu.make_async_copy(
            rows_v.at[b], out_hbm.at[wid, 0], ssem.at[b]).wait()

    for b in range(NBUF):
        start_gather(b, b)

    @pl.loop(0, NCHUNK, step=NBUF)
    def _(g0):
        for b in range(NBUF):
            g = g0 + b
            wait_gather(b)
            start_scatter(g, b)

            @pl.when(g + NBUF < NCHUNK)
            def _():
                wait_scatter(b)
                start_gather(g + NBUF, b)

    for b in range(NBUF):
        wait_scatter(b)


_emb_call = pl.kernel(
    _emb_body,
    out_type=jax.ShapeDtypeStruct((NW, NCHUNK, CHUNK, D), jnp.float32),
    mesh=plsc.VectorSubcoreMesh(core_axis_name="c", subcore_axis_name="s"),
    scratch_types=[
        pltpu.VMEM((NCHUNK, CHUNK), jnp.int32),
        pltpu.VMEM((NBUF, CHUNK, D), jnp.float32),
        pltpu.SemaphoreType.DMA((NBUF,)),
        pltpu.SemaphoreType.DMA((NBUF,)),
    ],
)


def _mask_body(tok_ref, m_ref):
    m_ref[...] = (tok_ref[...] > 0).astype(jnp.int8)


_mask_call = pl.pallas_call(
    _mask_body,
    out_shape=jax.ShapeDtypeStruct((BATCH, SEQLEN), jnp.int8),
)


def kernel(tokenid, table):
    idx3 = tokenid.reshape(NW, NCHUNK, CHUNK)
    emb = _emb_call(idx3, table)
    input_emb = emb.reshape(BATCH, SEQLEN, D)
    mask = _mask_call(tokenid).astype(jnp.bool_)
    return (input_emb, mask)


# speedup vs baseline: 8.0128x; 1.0240x over previous
import jax
import jax.numpy as jnp
from jax import lax
from jax.experimental import pallas as pl
from jax.experimental.pallas import tpu as pltpu
from jax.experimental.pallas import tpu_sc as plsc

BATCH = 1024
SEQLEN = 200
D = 128
B = BATCH * SEQLEN
NC = 2
NS = 16
NW = NC * NS
CHUNK = 128
NCHUNK = B // (NW * CHUNK)


NBUF = 5


def _emb_body(idx_hbm, table_hbm, out_hbm, idx_v, rows_v, gsem, ssem):
    wid = lax.axis_index("s") * NC + lax.axis_index("c")
    pltpu.sync_copy(idx_hbm.at[wid], idx_v)

    def start_gather(g, b):
        pltpu.make_async_copy(
            table_hbm.at[idx_v.at[g]], rows_v.at[b], gsem.at[b]).start()

    def wait_gather(b):
        pltpu.make_async_copy(
            table_hbm.at[idx_v.at[0]], rows_v.at[b], gsem.at[b]).wait()

    def start_scatter(g, b):
        pltpu.make_async_copy(
            rows_v.at[b], out_hbm.at[wid, g], ssem.at[b]).start()

    def wait_scatter(b):
        pltpu.make_async_copy(
            rows_v.at[b], out_hbm.at[wid, 0], ssem.at[b]).wait()

    for b in range(NBUF):
        start_gather(b, b)

    @pl.loop(0, NCHUNK, step=NBUF)
    def _(g0):
        for b in range(NBUF):
            g = g0 + b
            wait_gather(b)
            start_scatter(g, b)

            @pl.when(g + NBUF < NCHUNK)
            def _():
                wait_scatter(b)
                start_gather(g + NBUF, b)

    for b in range(NBUF):
        wait_scatter(b)


_emb_call = pl.kernel(
    _emb_body,
    out_type=jax.ShapeDtypeStruct((NW, NCHUNK, CHUNK, D), jnp.float32),
    mesh=plsc.VectorSubcoreMesh(core_axis_name="c", subcore_axis_name="s"),
    scratch_types=[
        pltpu.VMEM((NCHUNK, CHUNK), jnp.int32),
        pltpu.VMEM((NBUF, CHUNK, D), jnp.float32),
        pltpu.SemaphoreType.DMA((NBUF,)),
        pltpu.SemaphoreType.DMA((NBUF,)),
    ],
)


def _mask_body(tok_ref, m_ref):
    m_ref[...] = (tok_ref[...] > 0).astype(jnp.int8)


_mask_call = pl.pallas_call(
    _mask_body,
    out_shape=jax.ShapeDtypeStruct((BATCH, SEQLEN), jnp.int8),
)


def kernel(tokenid, table):
    idx3 = tokenid.reshape(NW, NCHUNK, CHUNK)
    emb = _emb_call(idx3, table)
    input_emb = emb.reshape(BATCH, SEQLEN, D)
    mask = _mask_call(tokenid).astype(jnp.bool_)
    return (input_emb, mask)
